# Initial kernel scaffold; baseline (speedup 1.0000x reference)
import jax
import jax.numpy as jnp
from jax.experimental import pallas as pl


def kernel(x, edge_index, edge_attr, W_edge, b_edge):
    raise NotImplementedError("write your pallas kernel here")


# speedup vs baseline: 3.1796x; 3.1796x over previous
import functools

import jax
import jax.numpy as jnp
from jax import lax
from jax.experimental import pallas as pl
from jax.experimental.pallas import tpu as pltpu
from jax.experimental.pallas import tpu_sc as plsc

N_NODES = 10000
N_EDGES = 320000
D_FEAT = 128
NUM_REL = 16

N_PAD = 10240
C = 128
NC, NS = 2, 16
EDGES_PER_CORE = N_EDGES // NC
CHUNKS_PER_CORE = EDGES_PER_CORE // C
ROWS_PER_TILE = N_PAD // NS


def _sc_body(src_hbm, dst_hbm, attr_hbm, xaug_hbm, px_hbm, pa_hbm,
             src_i, dst_i, idx_i, rows, abuf, acc, gsem):
    cid = lax.axis_index("c")
    sid = lax.axis_index("s")
    t0 = sid * ROWS_PER_TILE
    lanes = lax.iota(jnp.int32, 16)
    zvec = jnp.zeros((16,), jnp.float32)

    def _fill_iota(base):
        for k in range(C // 16):
            idx_i[pl.ds(k * 16, 16)] = base + k * 16 + lanes

    def _zero_rows(i, _):
        r = i // (D_FEAT // 16)
        c0 = (i % (D_FEAT // 16)) * 16
        rows[r, pl.ds(c0, 16)] = zvec
        return _

    def _zero_acc():
        for j in range(ROWS_PER_TILE // C):
            _fill_iota(t0 + j * C)
            pltpu.sync_copy(rows, acc.at[idx_i])

    def _drain(out_hbm):
        for j in range(ROWS_PER_TILE // C):
            r0 = t0 + j * C
            _fill_iota(r0)
            pltpu.sync_copy(acc.at[idx_i], rows)
            pltpu.sync_copy(rows, out_hbm.at[cid, pl.ds(r0, C)])

    n_chunks = 78 + jnp.where(sid < CHUNKS_PER_CORE - 78 * NS, 1, 0)

    lax.fori_loop(0, C * (D_FEAT // 16), _zero_rows, None)
    _zero_acc()
    plsc.subcore_barrier()

    def _chunk_x(i, _):
        chunk = sid + i * NS
        base = cid * EDGES_PER_CORE + chunk * C
        pltpu.sync_copy(src_hbm.at[pl.ds(base, C)], src_i)
        pltpu.sync_copy(dst_hbm.at[pl.ds(base, C)], dst_i)
        pltpu.async_copy(xaug_hbm.at[src_i], rows, gsem).wait()
        pltpu.sync_copy(rows, acc.at[dst_i], add=True)
        return _
    lax.fori_loop(0, n_chunks, _chunk_x, None)

    plsc.subcore_barrier()
    _drain(px_hbm)

    lax.fori_loop(0, C * (D_FEAT // 16), _zero_rows, None)
    _zero_acc()
    plsc.subcore_barrier()

    def _chunk_a(i, _):
        chunk = sid + i * NS
        base = cid * EDGES_PER_CORE + chunk * C
        pltpu.sync_copy(dst_hbm.at[pl.ds(base, C)], dst_i)
        pltpu.sync_copy(attr_hbm.at[pl.ds(base, C)], abuf)

        def _expand(e, _2):
            rows[e, pl.ds(0, NUM_REL)] = abuf[e, pl.ds(0, NUM_REL)]
            return _2
        lax.fori_loop(0, C, _expand, None)
        pltpu.sync_copy(rows, acc.at[dst_i], add=True)
        return _
    lax.fori_loop(0, n_chunks, _chunk_a, None)

    plsc.subcore_barrier()
    _drain(pa_hbm)


_sc_scatter = functools.partial(
    pl.kernel,
    out_type=(
        jax.ShapeDtypeStruct((NC, N_PAD, D_FEAT), jnp.float32),
        jax.ShapeDtypeStruct((NC, N_PAD, D_FEAT), jnp.float32),
    ),
    mesh=plsc.VectorSubcoreMesh(core_axis_name="c", subcore_axis_name="s"),
    scratch_types=[
        pltpu.VMEM((C,), jnp.int32),
        pltpu.VMEM((C,), jnp.int32),
        pltpu.VMEM((C,), jnp.int32),
        pltpu.VMEM((C, D_FEAT), jnp.float32),
        pltpu.VMEM((C, NUM_REL), jnp.float32),
        pltpu.VMEM_SHARED((N_PAD, D_FEAT), jnp.float32),
        pltpu.SemaphoreType.DMA,
    ],
)(_sc_body)


def _combine_body(px_ref, pa_ref, w_ref, o_ref):
    xs = px_ref[0] + px_ref[1]
    at = pa_ref[0] + pa_ref[1]
    o_ref[...] = xs + jnp.dot(at, w_ref[...],
                              preferred_element_type=jnp.float32)


_ROWS_BLK = 1000

_combine = pl.pallas_call(
    _combine_body,
    grid=(N_NODES // _ROWS_BLK,),
    in_specs=[
        pl.BlockSpec((NC, _ROWS_BLK, D_FEAT), lambda i: (0, i, 0)),
        pl.BlockSpec((NC, _ROWS_BLK, D_FEAT), lambda i: (0, i, 0)),
        pl.BlockSpec((D_FEAT, D_FEAT), lambda i: (0, 0)),
    ],
    out_specs=pl.BlockSpec((_ROWS_BLK, D_FEAT), lambda i: (i, 0)),
    out_shape=jax.ShapeDtypeStruct((N_NODES, D_FEAT), jnp.float32),
)


def kernel(x, edge_index, edge_attr, W_edge, b_edge):
    src = edge_index[0].astype(jnp.int32)
    dst = edge_index[1].astype(jnp.int32)
    x_aug = x + b_edge[None, :]
    w_pad = jnp.zeros((D_FEAT, D_FEAT), jnp.float32).at[:NUM_REL].set(W_edge)
    px, pa = _sc_scatter(src, dst, edge_attr, x_aug)
    return _combine(px, pa, w_pad)


# speedup vs baseline: 4.1432x; 1.3031x over previous
import functools

import jax
import jax.numpy as jnp
from jax import lax
from jax.experimental import pallas as pl
from jax.experimental.pallas import tpu as pltpu
from jax.experimental.pallas import tpu_sc as plsc

N_NODES = 10000
N_EDGES = 320000
D_FEAT = 128
NUM_REL = 16

N_PAD = 10112
C = 128
NC, NS = 2, 16
EDGES_PER_CORE = N_EDGES // NC
CHUNKS_PER_CORE = EDGES_PER_CORE // C
ROWS_PER_TILE = N_PAD // NS
RBLK = 128
TAIL = ROWS_PER_TILE - 4 * RBLK
MAX_CHUNKS = (CHUNKS_PER_CORE + NS - 1) // NS


def _sc_body(src_hbm, dst_hbm, attr_hbm, xaug_hbm, px_hbm, pa_hbm,
             src0, src1, dst0, dst1, idx_i, rows0, rows1, abuf,
             acc, g0, g1):
    cid = lax.axis_index("c")
    sid = lax.axis_index("s")
    t0 = sid * ROWS_PER_TILE
    lanes = lax.iota(jnp.int32, 16)
    zvec = jnp.zeros((16,), jnp.float32)

    def _fill_iota(base, limit=C):
        for k in range(C // 16):
            v = base + k * 16 + lanes
            if (k + 1) * 16 > limit:
                v = jnp.where(k * 16 + lanes >= limit, base, v)
            idx_i[pl.ds(k * 16, 16)] = v

    def _zero(buf):
        def _z(i, _):
            r = i // (D_FEAT // 16)
            c0 = (i % (D_FEAT // 16)) * 16
            buf[r, pl.ds(c0, 16)] = zvec
            return _
        lax.fori_loop(0, C * (D_FEAT // 16), _z, None)

    def _zero_acc():
        for j in range(4):
            _fill_iota(t0 + j * RBLK)
            pltpu.sync_copy(rows0, acc.at[idx_i])
        _fill_iota(t0 + 4 * RBLK, TAIL)
        pltpu.sync_copy(rows0, acc.at[idx_i])

    def _drain(out_hbm):
        for j in range(4):
            r0 = t0 + j * RBLK
            _fill_iota(r0)
            pltpu.sync_copy(acc.at[idx_i], rows0)
            pltpu.sync_copy(rows0, out_hbm.at[cid, pl.ds(r0, RBLK)])
        r0 = t0 + 4 * RBLK
        _fill_iota(r0, TAIL)
        pltpu.sync_copy(acc.at[idx_i], rows0)
        pltpu.sync_copy(rows0.at[pl.ds(0, TAIL)],
                        out_hbm.at[cid, pl.ds(r0, TAIL)])

    n_chunks = 78 + jnp.where(sid < CHUNKS_PER_CORE - 78 * NS, 1, 0)

    def _ebase(i):
        return cid * EDGES_PER_CORE + (sid + i * NS) * C

    _zero(rows0)
    _zero_acc()
    plsc.subcore_barrier()

    def _load_idx(i, s_ref, d_ref):
        base = _ebase(i)
        pltpu.sync_copy(src_hbm.at[pl.ds(base, C)], s_ref)
        pltpu.sync_copy(dst_hbm.at[pl.ds(base, C)], d_ref)

    _load_idx(0, src0, dst0)
    cp0 = pltpu.async_copy(xaug_hbm.at[src0], rows0, g0)

    def _outer(i2, _):
        i0 = 2 * i2
        i1 = 2 * i2 + 1

        @pl.when(i1 < n_chunks)
        def _start1():
            _load_idx(i1, src1, dst1)
            pltpu.async_copy(xaug_hbm.at[src1], rows1, g1)

        @pl.when(i0 < n_chunks)
        def _fin0():
            cp0.wait()
            pltpu.sync_copy(rows0, acc.at[dst0], add=True)

        @pl.when(i0 + 2 < n_chunks)
        def _start0():
            _load_idx(i0 + 2, src0, dst0)
            pltpu.async_copy(xaug_hbm.at[src0], rows0, g0)

        @pl.when(i1 < n_chunks)
        def _fin1():
            pltpu.make_async_copy(xaug_hbm.at[src1], rows1, g1).wait()
            pltpu.sync_copy(rows1, acc.at[dst1], add=True)
        return _
    lax.fori_loop(0, (MAX_CHUNKS + 1) // 2, _outer, None)

    plsc.subcore_barrier()
    _drain(px_hbm)

    _zero(rows0)
    _zero(rows1)
    _zero_acc()
    plsc.subcore_barrier()

    def _load_expand(i, d_ref, r_ref):
        base = _ebase(i)
        pltpu.sync_copy(dst_hbm.at[pl.ds(base, C)], d_ref)
        pltpu.sync_copy(attr_hbm.at[pl.ds(base, C)], abuf)

        def _expand(e, _2):
            r_ref[e, pl.ds(0, NUM_REL)] = abuf[e, pl.ds(0, NUM_REL)]
            return _2
        lax.fori_loop(0, C, _expand, None)

    _load_expand(0, dst0, rows0)
    sp0 = pltpu.async_copy(rows0, acc.at[dst0], g0, add=True)

    def _outer_a(i2, _):
        i0 = 2 * i2
        i1 = 2 * i2 + 1

        @pl.when(i1 < n_chunks)
        def _s1():
            _load_expand(i1, dst1, rows1)
            pltpu.async_copy(rows1, acc.at[dst1], g1, add=True)

        @pl.when(i0 < n_chunks)
        def _w0():
            sp0.wait()

        @pl.when(i0 + 2 < n_chunks)
        def _s0():
            _load_expand(i0 + 2, dst0, rows0)
            pltpu.async_copy(rows0, acc.at[dst0], g0, add=True)

        @pl.when(i1 < n_chunks)
        def _w1():
            pltpu.make_async_copy(rows1, acc.at[dst1], g1).wait()
        return _
    lax.fori_loop(0, (MAX_CHUNKS + 1) // 2, _outer_a, None)

    plsc.subcore_barrier()
    _drain(pa_hbm)


_sc_scatter = functools.partial(
    pl.kernel,
    out_type=(
        jax.ShapeDtypeStruct((NC, N_PAD, D_FEAT), jnp.float32),
        jax.ShapeDtypeStruct((NC, N_PAD, D_FEAT), jnp.float32),
    ),
    mesh=plsc.VectorSubcoreMesh(core_axis_name="c", subcore_axis_name="s"),
    scratch_types=[
        pltpu.VMEM((C,), jnp.int32),
        pltpu.VMEM((C,), jnp.int32),
        pltpu.VMEM((C,), jnp.int32),
        pltpu.VMEM((C,), jnp.int32),
        pltpu.VMEM((C,), jnp.int32),
        pltpu.VMEM((C, D_FEAT), jnp.float32),
        pltpu.VMEM((C, D_FEAT), jnp.float32),
        pltpu.VMEM((C, NUM_REL), jnp.float32),
        pltpu.VMEM_SHARED((N_PAD, D_FEAT), jnp.float32),
        pltpu.SemaphoreType.DMA,
        pltpu.SemaphoreType.DMA,
    ],
)(_sc_body)


def _combine_body(px_ref, pa_ref, w_ref, o_ref):
    xs = px_ref[0] + px_ref[1]
    at = pa_ref[0] + pa_ref[1]
    o_ref[...] = xs + jnp.dot(at, w_ref[...],
                              preferred_element_type=jnp.float32)


_ROWS_BLK = 1000

_combine = pl.pallas_call(
    _combine_body,
    grid=(N_NODES // _ROWS_BLK,),
    in_specs=[
        pl.BlockSpec((NC, _ROWS_BLK, D_FEAT), lambda i: (0, i, 0)),
        pl.BlockSpec((NC, _ROWS_BLK, D_FEAT), lambda i: (0, i, 0)),
        pl.BlockSpec((D_FEAT, D_FEAT), lambda i: (0, 0)),
    ],
    out_specs=pl.BlockSpec((_ROWS_BLK, D_FEAT), lambda i: (i, 0)),
    out_shape=jax.ShapeDtypeStruct((N_NODES, D_FEAT), jnp.float32),
)


def kernel(x, edge_index, edge_attr, W_edge, b_edge):
    src = edge_index[0].astype(jnp.int32)
    dst = edge_index[1].astype(jnp.int32)
    x_aug = x + b_edge[None, :]
    w_pad = jnp.zeros((D_FEAT, D_FEAT), jnp.float32).at[:NUM_REL].set(W_edge)
    px, pa = _sc_scatter(src, dst, edge_attr, x_aug)
    return _combine(px, pa, w_pad)
